# Initial kernel scaffold; baseline (speedup 1.0000x reference)
import jax
import jax.numpy as jnp
from jax.experimental import pallas as pl


def kernel(boxes, scores, labels):
    raise NotImplementedError("write your pallas kernel here")


# speedup vs baseline: 1.0294x; 1.0294x over previous
import functools

import jax
import jax.numpy as jnp
from jax.experimental import pallas as pl

_SCORE_THRESH = 0.05
_NMS_THRESH = 0.5
_DET_PER_IMG = 300
_TOPK = 512
_EPS = 1e-07
_NEG_INF = float("-inf")
_U = (-0.375, -0.125, 0.125, 0.375)


def _cumsum_lanes(x, npad):
    sh = 1
    while sh < npad:
        shifted = jnp.concatenate(
            [jnp.zeros((1, sh), x.dtype), x[:, : npad - sh]], axis=1)
        x = x + shifted
        sh *= 2
    return x


def _nms_body(inp_ref, out_ref, *, npad):
    f32 = jnp.float32
    data = inp_ref[0]
    s_row = data[5:6, :]
    valid = s_row > _SCORE_THRESH
    key = jnp.where(valid, jax.lax.bitcast_convert_type(s_row, jnp.int32),
                    jnp.int32(-1))

    def bs_body(_, carry):
        lo, hi = carry
        mid = lo + (hi - lo) // 2
        cnt = jnp.sum((key >= mid).astype(f32))
        big = cnt >= float(_TOPK)
        return (jnp.where(big, mid, lo), jnp.where(big, hi, mid))

    lo, _ = jax.lax.fori_loop(
        0, 31, bs_body, (jnp.int32(-2), jnp.int32(1 << 30)))
    v = lo
    c_gt = jnp.sum((key > v).astype(f32))
    quota = jnp.int32(_TOPK) - c_gt.astype(jnp.int32)
    eq = key == v
    eq_i = eq.astype(jnp.int32)
    eq_rank = _cumsum_lanes(eq_i, npad) - eq_i
    selected = (key > v) | (eq & (eq_rank < quota))
    sel_i = selected.astype(jnp.int32)
    rank = _cumsum_lanes(sel_i, npad) - sel_i

    k = _TOPK
    iota0 = jax.lax.broadcasted_iota(jnp.int32, (k, k), 0)
    acc = jnp.zeros((8, k), f32)
    for blk in range(npad // k):
        sl = slice(blk * k, (blk + 1) * k)
        oh = ((iota0 == rank[:, sl]) & selected[:, sl]).astype(f32)
        acc = acc + jax.lax.dot_general(
            data[:, sl], oh, (((1,), (1,)), ((), ())),
            preferred_element_type=f32, precision=jax.lax.Precision.HIGHEST)
    g = acc

    eyef = (iota0 == jax.lax.broadcasted_iota(jnp.int32, (k, k), 1)).astype(f32)
    gt = jax.lax.dot_general(eyef, g, (((1,), (1,)), ((), ())),
                             preferred_element_type=f32, precision=jax.lax.Precision.HIGHEST)

    cxi, cyi = gt[:, 0:1], gt[:, 1:2]
    wi, hi, ai = gt[:, 2:3], gt[:, 3:4], gt[:, 4:5]
    si, li, ii = gt[:, 5:6], gt[:, 6:7], gt[:, 7:8]
    cxj, cyj = g[0:1, :], g[1:2, :]
    wj, hj, aj = g[2:3, :], g[3:4, :], g[4:5, :]
    sj, lj, ij = g[5:6, :], g[6:7, :], g[7:8, :]

    cai, sai = jnp.cos(ai), jnp.sin(ai)
    caj, saj = jnp.cos(aj), jnp.sin(aj)
    whalf, hhalf = wj * 0.5, hj * 0.5

    cnt = jnp.zeros((k, k), f32)
    for sidx in range(16):
        ox = _U[sidx % 4] * wi
        oy = _U[sidx // 4] * hi
        px = cxi + ox * cai - oy * sai
        py = cyi + ox * sai + oy * cai
        dx = px - cxj
        dy = py - cyj
        lx = dx * caj + dy * saj
        ly = -dx * saj + dy * caj
        sd = jnp.maximum(jnp.abs(lx) - whalf, jnp.abs(ly) - hhalf)
        cnt = cnt + (sd <= 0.0).astype(f32)
    frac = cnt * (1.0 / 16.0)
    fract = jax.lax.dot_general(frac, eyef, (((0,), (0,)), ((), ())),
                                preferred_element_type=f32, precision=jax.lax.Precision.HIGHEST)

    area_i = wi * hi
    area_j = wj * hj
    inter = 0.5 * (area_i * frac + area_j * fract)
    iou = inter / (area_i + area_j - inter + _EPS)

    validj = sj > _SCORE_THRESH
    stronger = (sj > si) | ((sj == si) & (ij < ii))
    m = stronger & (lj == li) & validj
    max_iou = jnp.max(jnp.where(m, iou, 0.0), axis=1, keepdims=True)
    keep = (max_iou <= _NMS_THRESH) & (si > _SCORE_THRESH)
    keep_f = keep.astype(f32)
    keep_row = jax.lax.dot_general(keep_f, eyef, (((0,), (0,)), ((), ())),
                                   preferred_element_type=f32, precision=jax.lax.Precision.HIGHEST)

    ks_col = jnp.where(keep, si, _NEG_INF)
    ks_row = jnp.where(keep_row > 0.0, sj, _NEG_INF)
    better = (ks_row > ks_col) | ((ks_row == ks_col) & (ij < ii))
    rank2 = jnp.sum(better.astype(f32), axis=1, keepdims=True)
    rank2_row = jax.lax.dot_general(rank2, eyef, (((0,), (0,)), ((), ())),
                                    preferred_element_type=f32, precision=jax.lax.Precision.HIGHEST)
    oh2 = (iota0 == rank2_row.astype(jnp.int32)).astype(f32)

    m8 = jnp.broadcast_to(keep_row, (8, k))
    rowsel6 = (jax.lax.broadcasted_iota(jnp.int32, (8, k), 0) == 6).astype(f32)
    gm = g * m8 + rowsel6 * (m8 - 1.0)
    out_ref[0] = jax.lax.dot_general(gm, oh2, (((1,), (1,)), ((), ())),
                                     preferred_element_type=f32, precision=jax.lax.Precision.HIGHEST)


def kernel(boxes, scores, labels):
    b, n, _ = boxes.shape
    f32 = jnp.float32
    npad = -(-n // _TOPK) * _TOPK
    bt = jnp.transpose(boxes, (0, 2, 1)).astype(f32)
    bt = jnp.pad(bt, ((0, 0), (0, 0), (0, npad - n)))
    sc = jnp.pad(scores.astype(f32)[:, None, :],
                 ((0, 0), (0, 0), (0, npad - n)), constant_values=-1.0)
    lb = jnp.pad(labels.astype(f32)[:, None, :],
                 ((0, 0), (0, 0), (0, npad - n)))
    idx = jnp.broadcast_to(jnp.arange(npad, dtype=f32)[None, None, :],
                           (b, 1, npad))
    packed = jnp.concatenate([bt, sc, lb, idx], axis=1)

    out = pl.pallas_call(
        functools.partial(_nms_body, npad=npad),
        grid=(b,),
        in_specs=[pl.BlockSpec((1, 8, npad), lambda i: (i, 0, 0))],
        out_specs=pl.BlockSpec((1, 8, _TOPK), lambda i: (i, 0, 0)),
        out_shape=jax.ShapeDtypeStruct((b, 8, _TOPK), f32),
    )(packed)

    det = out[:, :, :_DET_PER_IMG]
    out_boxes = jnp.transpose(det[:, 0:5, :], (0, 2, 1))
    out_scores = det[:, 5, :]
    out_labels = det[:, 6, :].astype(jnp.int32)
    return out_boxes, out_scores, out_labels


# speedup vs baseline: 1.6833x; 1.6352x over previous
import functools

import jax
import jax.numpy as jnp
from jax.experimental import pallas as pl

_SCORE_THRESH = 0.05
_NMS_THRESH = 0.5
_DET_PER_IMG = 300
_TOPK = 512
_EPS = 1e-07
_NEG_INF = float("-inf")
_U = (-0.375, -0.125, 0.125, 0.375)


def _cumsum_lanes(x, npad):
    sh = 1
    while sh < npad:
        shifted = jnp.concatenate(
            [jnp.zeros((1, sh), x.dtype), x[:, : npad - sh]], axis=1)
        x = x + shifted
        sh *= 2
    return x


def _split3(x):
    f32 = jnp.float32
    hi = x.astype(jnp.bfloat16)
    r1 = x - hi.astype(f32)
    mid = r1.astype(jnp.bfloat16)
    lo = (r1 - mid.astype(f32)).astype(jnp.bfloat16)
    return hi, mid, lo


def _dot_lhs_exact(lhs3, rhs_bf, dims):
    f32 = jnp.float32
    hi, mid, lo = lhs3
    d1 = jax.lax.dot_general(hi, rhs_bf, dims, preferred_element_type=f32)
    d2 = jax.lax.dot_general(mid, rhs_bf, dims, preferred_element_type=f32)
    d3 = jax.lax.dot_general(lo, rhs_bf, dims, preferred_element_type=f32)
    return (d1 + d2) + d3


def _dot_rhs_exact(lhs_bf, rhs, dims):
    f32 = jnp.float32
    hi, mid, lo = _split3(rhs)
    d1 = jax.lax.dot_general(lhs_bf, hi, dims, preferred_element_type=f32)
    d2 = jax.lax.dot_general(lhs_bf, mid, dims, preferred_element_type=f32)
    d3 = jax.lax.dot_general(lhs_bf, lo, dims, preferred_element_type=f32)
    return (d1 + d2) + d3


def _nms_body(inp_ref, out_ref, *, npad):
    f32 = jnp.float32
    bf = jnp.bfloat16
    data = inp_ref[0]
    s_row = data[5:6, :]
    valid = s_row > _SCORE_THRESH
    key = jnp.where(valid, jax.lax.bitcast_convert_type(s_row, jnp.int32),
                    jnp.int32(-1))

    lo = jnp.int32(-2)
    span = 1 << 30
    for _ in range(10):
        step = span // 8
        t = jnp.int32(0)
        for p in range(1, 8):
            cnt = jnp.sum((key >= lo + p * step).astype(f32))
            t = t + (cnt >= float(_TOPK)).astype(jnp.int32)
        lo = lo + t * step
        span = step
    v = lo
    c_gt = jnp.sum((key > v).astype(f32))
    quota = jnp.int32(_TOPK) - c_gt.astype(jnp.int32)
    eq = key == v
    eq_i = eq.astype(jnp.int32)
    eq_rank = _cumsum_lanes(eq_i, npad) - eq_i
    selected = (key > v) | (eq & (eq_rank < quota))
    sel_i = selected.astype(jnp.int32)
    rank = _cumsum_lanes(sel_i, npad) - sel_i

    k = _TOPK
    iota0 = jax.lax.broadcasted_iota(jnp.int32, (k, k), 0)
    data3 = _split3(data)
    acc = jnp.zeros((8, k), f32)
    dims_nn = (((1,), (1,)), ((), ()))
    for blk in range(npad // k):
        sl = slice(blk * k, (blk + 1) * k)
        oh = ((iota0 == rank[:, sl]) & selected[:, sl]).astype(bf)
        acc = acc + _dot_lhs_exact(
            tuple(d[:, sl] for d in data3), oh, dims_nn)
    g = acc

    eyeb = (iota0 == jax.lax.broadcasted_iota(jnp.int32, (k, k), 1)).astype(bf)
    aj = g[4:5, :]
    caj, saj = jnp.cos(aj), jnp.sin(aj)
    g10 = jnp.concatenate([g, caj, saj], axis=0)
    gt = _dot_rhs_exact(eyeb, g10, dims_nn)

    cxi, cyi = gt[:, 0:1], gt[:, 1:2]
    wi, hi = gt[:, 2:3], gt[:, 3:4]
    si, li, ii = gt[:, 5:6], gt[:, 6:7], gt[:, 7:8]
    cai, sai = gt[:, 8:9], gt[:, 9:10]
    cxj, cyj = g[0:1, :], g[1:2, :]
    wj, hj = g[2:3, :], g[3:4, :]
    sj, lj, ij = g[5:6, :], g[6:7, :], g[7:8, :]
    whalf, hhalf = wj * 0.5, hj * 0.5

    cnt = jnp.zeros((k, k), f32)
    for sidx in range(16):
        ox = _U[sidx % 4] * wi
        oy = _U[sidx // 4] * hi
        px = cxi + ox * cai - oy * sai
        py = cyi + ox * sai + oy * cai
        dx = px - cxj
        dy = py - cyj
        lx = dx * caj + dy * saj
        ly = -dx * saj + dy * caj
        inside = (jnp.abs(lx) <= whalf) & (jnp.abs(ly) <= hhalf)
        cnt = cnt + inside.astype(f32)
    frac = cnt * (1.0 / 16.0)
    fract = jax.lax.dot_general(frac.astype(bf), eyeb, (((0,), (0,)), ((), ())),
                                preferred_element_type=f32)

    area_i = wi * hi
    area_j = wj * hj
    inter = 0.5 * (area_i * frac + area_j * fract)
    iou = inter / (area_i + area_j - inter + _EPS)

    validj = sj > _SCORE_THRESH
    stronger = (sj > si) | ((sj == si) & (ij < ii))
    m = stronger & (lj == li) & validj
    max_iou = jnp.max(jnp.where(m, iou, 0.0), axis=1, keepdims=True)
    keep = (max_iou <= _NMS_THRESH) & (si > _SCORE_THRESH)
    keep_f = keep.astype(bf)
    keep_row = jax.lax.dot_general(keep_f, eyeb, (((0,), (0,)), ((), ())),
                                   preferred_element_type=f32)

    ks_col = jnp.where(keep, si, _NEG_INF)
    ks_row = jnp.where(keep_row > 0.0, sj, _NEG_INF)
    worse = (ks_col > ks_row) | ((ks_col == ks_row) & (ii < ij))
    rank2_row = jnp.sum(worse.astype(f32), axis=0, keepdims=True)
    oh2 = (iota0 == rank2_row.astype(jnp.int32)).astype(bf)

    m8 = jnp.broadcast_to(keep_row, (8, k))
    rowsel6 = (jax.lax.broadcasted_iota(jnp.int32, (8, k), 0) == 6).astype(f32)
    gm = g * m8 + rowsel6 * (m8 - 1.0)
    out_ref[0] = _dot_lhs_exact(_split3(gm), oh2, dims_nn)


def kernel(boxes, scores, labels):
    b, n, _ = boxes.shape
    f32 = jnp.float32
    npad = -(-n // _TOPK) * _TOPK
    bt = jnp.transpose(boxes, (0, 2, 1)).astype(f32)
    bt = jnp.pad(bt, ((0, 0), (0, 0), (0, npad - n)))
    sc = jnp.pad(scores.astype(f32)[:, None, :],
                 ((0, 0), (0, 0), (0, npad - n)), constant_values=-1.0)
    lb = jnp.pad(labels.astype(f32)[:, None, :],
                 ((0, 0), (0, 0), (0, npad - n)))
    idx = jnp.broadcast_to(jnp.arange(npad, dtype=f32)[None, None, :],
                           (b, 1, npad))
    packed = jnp.concatenate([bt, sc, lb, idx], axis=1)

    out = pl.pallas_call(
        functools.partial(_nms_body, npad=npad),
        grid=(b,),
        in_specs=[pl.BlockSpec((1, 8, npad), lambda i: (i, 0, 0))],
        out_specs=pl.BlockSpec((1, 8, _TOPK), lambda i: (i, 0, 0)),
        out_shape=jax.ShapeDtypeStruct((b, 8, _TOPK), f32),
    )(packed)

    det = out[:, :, :_DET_PER_IMG]
    out_boxes = jnp.transpose(det[:, 0:5, :], (0, 2, 1))
    out_scores = det[:, 5, :]
    out_labels = det[:, 6, :].astype(jnp.int32)
    return out_boxes, out_scores, out_labels
